# Initial kernel scaffold; baseline (speedup 1.0000x reference)
import jax
import jax.numpy as jnp
from jax.experimental import pallas as pl


def kernel(x, edge_index, batch, params):
    raise NotImplementedError("write your pallas kernel here")


# speedup vs baseline: 5.7648x; 5.7648x over previous
import functools

import jax
import jax.numpy as jnp
from jax import lax
from jax.experimental import pallas as pl
from jax.experimental.pallas import tpu as pltpu
from jax.experimental.pallas import tpu_sc as plsc

N_NODES = 10000
N_EDGES = 320000
D_FEAT = 128
NUM_GRAPHS = 64

_NC = 2
_NS = 16
_NW = _NC * _NS
_EDGES_PER_W = N_EDGES // _NW
_CHUNK = 128
_NFULL = _EDGES_PER_W // _CHUNK
_REM = _EDGES_PER_W - _NFULL * _CHUNK
_ROWS_PER_TILE = 624
_ROWS_TAIL = N_NODES - _NS * _ROWS_PER_TILE


def _segsum_body(h_hbm, src_hbm, dst_hbm, zeros_hbm, out_hbm,
                 srcv, dstv, rows, srcv_r, dstv_r, rows_r, acc, sem):
    cid = lax.axis_index("c")
    sid = lax.axis_index("s")
    wid = sid * _NC + cid

    pltpu.sync_copy(zeros_hbm, acc.at[pl.ds(sid * _ROWS_PER_TILE, _ROWS_PER_TILE)])

    @pl.when(sid == _NS - 1)
    def _():
        pltpu.sync_copy(zeros_hbm.at[pl.ds(0, _ROWS_TAIL)],
                        acc.at[pl.ds(_NS * _ROWS_PER_TILE, _ROWS_TAIL)])

    plsc.subcore_barrier()

    base = wid * _EDGES_PER_W

    def chunk_body(t, carry):
        off = base + t * _CHUNK
        pltpu.sync_copy(src_hbm.at[pl.ds(off, _CHUNK)], srcv)
        pltpu.async_copy(h_hbm.at[srcv], rows, sem).wait()
        pltpu.sync_copy(dst_hbm.at[pl.ds(off, _CHUNK)], dstv)
        pltpu.sync_copy(rows, acc.at[dstv], add=True)
        return carry

    lax.fori_loop(0, _NFULL, chunk_body, 0)

    off = base + _NFULL * _CHUNK
    pltpu.sync_copy(src_hbm.at[pl.ds(off, _REM)], srcv_r)
    pltpu.async_copy(h_hbm.at[srcv_r], rows_r, sem).wait()
    pltpu.sync_copy(dst_hbm.at[pl.ds(off, _REM)], dstv_r)
    pltpu.sync_copy(rows_r, acc.at[dstv_r], add=True)

    plsc.subcore_barrier()

    r0 = sid * _ROWS_PER_TILE
    pltpu.sync_copy(acc.at[pl.ds(r0, _ROWS_PER_TILE)],
                    out_hbm.at[cid, pl.ds(r0, _ROWS_PER_TILE)])

    @pl.when(sid == _NS - 1)
    def _():
        t0 = _NS * _ROWS_PER_TILE
        pltpu.sync_copy(acc.at[pl.ds(t0, _ROWS_TAIL)],
                        out_hbm.at[cid, pl.ds(t0, _ROWS_TAIL)])


_segsum = pl.kernel(
    _segsum_body,
    out_type=jax.ShapeDtypeStruct((_NC, N_NODES, D_FEAT), jnp.float32),
    mesh=plsc.VectorSubcoreMesh(core_axis_name="c", subcore_axis_name="s"),
    scratch_types=[
        pltpu.VMEM((_CHUNK,), jnp.int32),
        pltpu.VMEM((_CHUNK,), jnp.int32),
        pltpu.VMEM((_CHUNK, D_FEAT), jnp.float32),
        pltpu.VMEM((_REM,), jnp.int32),
        pltpu.VMEM((_REM,), jnp.int32),
        pltpu.VMEM((_REM, D_FEAT), jnp.float32),
        pltpu.VMEM_SHARED((N_NODES, D_FEAT), jnp.float32),
        pltpu.SemaphoreType.DMA,
    ],
)


def _mlp_body(h_ref, agg_ref, batch_ref, w1_ref, b1_ref, g_ref, be_ref,
              w2_ref, b2_ref, hout_ref, pooled_ref):
    h = h_ref[...]
    out = h + agg_ref[0] + agg_ref[1]
    out = jnp.dot(out, w1_ref[...], preferred_element_type=jnp.float32) + b1_ref[...]
    mean = jnp.mean(out, axis=0, keepdims=True)
    var = jnp.mean(jnp.square(out - mean), axis=0, keepdims=True)
    out = (out - mean) * lax.rsqrt(var + 1e-5) * g_ref[...] + be_ref[...]
    out = jnp.maximum(out, 0.0)
    out = jnp.dot(out, w2_ref[...], preferred_element_type=jnp.float32) + b2_ref[...]
    h2 = jnp.maximum(out, 0.0)
    hout_ref[...] = h2

    gids = lax.broadcasted_iota(jnp.int32, (N_NODES, NUM_GRAPHS), 1)
    mask = (batch_ref[...] == gids).astype(jnp.float32)
    sums = lax.dot_general(mask, h2, (((0,), (0,)), ((), ())),
                           preferred_element_type=jnp.float32)
    counts = jnp.sum(mask, axis=0)[:, None]
    pooled_ref[...] = sums / jnp.maximum(counts, 1.0)


_mlp = pl.pallas_call(
    _mlp_body,
    out_shape=(
        jax.ShapeDtypeStruct((N_NODES, D_FEAT), jnp.float32),
        jax.ShapeDtypeStruct((NUM_GRAPHS, D_FEAT), jnp.float32),
    ),
)


def _proj_body(p0_ref, p1_ref, p2_ref, wp_ref, bp_ref, wc_ref, bc_ref,
               z_ref, logits_ref):
    hjk = jnp.concatenate([p0_ref[...], p1_ref[...], p2_ref[...]], axis=1)
    z = jnp.dot(hjk, wp_ref[...], preferred_element_type=jnp.float32) + bp_ref[...]
    z_ref[...] = z
    logits_ref[...] = (
        jnp.dot(z, wc_ref[...], preferred_element_type=jnp.float32) + bc_ref[...])


def kernel(x, edge_index, batch, params):
    src = edge_index[0]
    dst = edge_index[1]
    batch2 = batch[:, None]
    zeros = jnp.zeros((_ROWS_PER_TILE, D_FEAT), jnp.float32)

    h = x
    pooled = []
    for i in range(3):
        p = params['conv%d' % i]
        agg = _segsum(h, src, dst, zeros)
        h, pool = _mlp(h, agg, batch2,
                       p['W1'], p['b1'][None, :], p['gamma'][None, :],
                       p['beta'][None, :], p['W2'], p['b2'][None, :])
        pooled.append(pool)

    proj = pl.pallas_call(
        _proj_body,
        out_shape=(
            jax.ShapeDtypeStruct((NUM_GRAPHS, params['Wp'].shape[1]), jnp.float32),
            jax.ShapeDtypeStruct((NUM_GRAPHS, params['Wc'].shape[1]), jnp.float32),
        ),
    )
    z, logits = proj(pooled[0], pooled[1], pooled[2],
                     params['Wp'], params['bp'][None, :],
                     params['Wc'], params['bc'][None, :])
    return z, logits
